# baseline (speedup 1.0000x reference)
import functools

import jax
import jax.numpy as jnp
from jax import lax
from jax.experimental import pallas as pl
from jax.experimental.pallas import tpu as pltpu
from jax.experimental.pallas import tpu_sc as plsc

N = 10000
NH = 256
E = 160000

NC = 2
NS = 16
NW = NC * NS

CHUNK = 3200
SUBB = 32
NRING = 8
ROWS_BASE = 312
BIG_WORKERS = 2
ROWS_BIG = 320
ACC_ROWS = ROWS_BIG

NHP = NH // 2

_PERM = [0] * NH
for _k in range(NH // 32):
    for _i in range(16):
        _PERM[32 * _k + 2 * _i] = 32 * _k + _i
        _PERM[32 * _k + 2 * _i + 1] = 32 * _k + 16 + _i
_PERM = tuple(_PERM)


def _pack_table(t):
    tb = t[:, jnp.array(_PERM, dtype=jnp.int32)].astype(jnp.bfloat16)
    return lax.bitcast_convert_type(tb.reshape(t.shape[0], NHP, 2),
                                    jnp.int32)


def _sc_spmm_body(dst_hbm, src_hbm, val_hbm, table_hbm, out_hbm,
                  dbuf, sbuf, vbuf, rows0, rows1, rows2, rows3,
                  rows4, rows5, rows6, rows7, acc,
                  sem, gsem0, gsem1, gsem2, gsem3,
                  gsem4, gsem5, gsem6, gsem7):
    rows_bufs = (rows0, rows1, rows2, rows3, rows4, rows5, rows6, rows7)
    gsems = (gsem0, gsem1, gsem2, gsem3, gsem4, gsem5, gsem6, gsem7)
    wid = lax.axis_index("c") * NS + lax.axis_index("s")
    lo = wid * ROWS_BASE + 8 * jnp.minimum(wid, BIG_WORKERS)
    hi = lo + jnp.where(wid < BIG_WORKERS, ROWS_BIG, ROWS_BASE)

    zf = jnp.zeros((16,), jnp.float32)
    zi = jnp.zeros((16,), jnp.int32)

    def zero_body(r, _):
        for k in range(NH // 16):
            acc[r, pl.ds(k * 16, 16)] = zf
        return 0
    lax.fori_loop(0, ACC_ROWS, zero_body, 0)

    pad_idx = jnp.full((16,), 1, jnp.int32) * lo
    def zero_idx(i, _):
        sbuf[pl.ds(i * 16, 16)] = pad_idx
        return 0
    lax.fori_loop(0, (CHUNK + 16) // 16, zero_idx, 0)

    def chunk_body(ch, _):
        off = ch * CHUNK
        c1 = pltpu.async_copy(dst_hbm.at[pl.ds(off, CHUNK)],
                              dbuf.at[pl.ds(0, CHUNK)], sem)
        c2 = pltpu.async_copy(src_hbm.at[pl.ds(off, CHUNK)],
                              sbuf.at[pl.ds(0, CHUNK)], sem)
        c3 = pltpu.async_copy(val_hbm.at[pl.ds(off, CHUNK)],
                              vbuf.at[pl.ds(0, CHUNK)], sem)
        c1.wait()
        c2.wait()
        c3.wait()

        def comp_body(j, cnt):
            base = j * 32
            d1 = dbuf[pl.ds(base, 16)]
            s1 = sbuf[pl.ds(base, 16)]
            v1 = vbuf[pl.ds(base, 16)]
            d2 = dbuf[pl.ds(base + 16, 16)]
            s2 = sbuf[pl.ds(base + 16, 16)]
            v2 = vbuf[pl.ds(base + 16, 16)]
            m1 = (d1 >= lo) & (d1 < hi)
            m2 = (d2 >= lo) & (d2 < hi)
            ps1 = plsc.cumsum(m1.astype(jnp.int32))
            ps2 = plsc.cumsum(m2.astype(jnp.int32))
            t1 = ps1[15]
            pos1 = cnt + ps1 - 1
            pos2 = cnt + t1 + ps2 - 1
            plsc.store_scatter(dbuf, [pos1], d1 - lo, mask=m1)
            plsc.store_scatter(sbuf, [pos1], s1, mask=m1)
            plsc.store_scatter(vbuf, [pos1], v1, mask=m1)
            plsc.store_scatter(dbuf, [pos2], d2 - lo, mask=m2)
            plsc.store_scatter(sbuf, [pos2], s2, mask=m2)
            plsc.store_scatter(vbuf, [pos2], v2, mask=m2)
            return cnt + t1 + ps2[15]
        cnt = lax.fori_loop(0, CHUNK // 32, comp_body, 0)

        dbuf[pl.ds(cnt, 16)] = zi
        sbuf[pl.ds(cnt, 16)] = pad_idx
        vbuf[pl.ds(cnt, 16)] = zf

        nbatch = (cnt + SUBB - 1) // SUBB
        nwave = (nbatch + NRING - 1) // NRING

        def wave_body(w, _):
            cps = []
            for r in range(NRING):
                b = w * NRING + r
                cp = pltpu.make_async_copy(
                    table_hbm.at[sbuf.at[pl.ds(b * SUBB, SUBB)]],
                    rows_bufs[r], gsems[r])
                cps.append(cp)

                @pl.when(b < nbatch)
                def _():
                    cp.start()

            for r in range(NRING):
                b = w * NRING + r

                @pl.when(b < nbatch)
                def _():
                    cps[r].wait()
                nedge = jnp.clip(cnt - b * SUBB, 0, SUBB)
                rbuf = rows_bufs[r]

                def e_body(i, _):
                    dle = dbuf[pl.ds(b * SUBB + i, 16)][0]
                    vv = jnp.full((16,), vbuf[pl.ds(b * SUBB + i, 16)][0],
                                  jnp.float32)
                    for k in range(NH // 32):
                        w = rbuf[i, pl.ds(k * 16, 16)]
                        pa, pb = plsc.unpack(
                            plsc.bitcast(w, jnp.bfloat16),
                            format=plsc.PackFormat.INTERLEAVED)
                        plsc.addupdate(acc.at[dle, pl.ds(k * 32, 16)],
                                       pa * vv)
                        plsc.addupdate(acc.at[dle, pl.ds(k * 32 + 16, 16)],
                                       pb * vv)
                    return 0
                lax.fori_loop(0, nedge, e_body, 0)
            return 0
        lax.fori_loop(0, nwave, wave_body, 0)
        return 0

    lax.fori_loop(0, E // CHUNK, chunk_body, 0)

    @pl.when(wid < BIG_WORKERS)
    def _():
        pltpu.sync_copy(acc.at[pl.ds(0, ROWS_BIG), :],
                        out_hbm.at[pl.ds(lo, ROWS_BIG), :])

    @pl.when(wid >= BIG_WORKERS)
    def _():
        pltpu.sync_copy(acc.at[pl.ds(0, ROWS_BASE), :],
                        out_hbm.at[pl.ds(lo, ROWS_BASE), :])


@functools.partial(jax.jit, static_argnums=())
def _spmm_sc(dst, src, val, table):
    mesh = plsc.VectorSubcoreMesh(core_axis_name="c", subcore_axis_name="s")
    f = pl.kernel(
        _sc_spmm_body,
        out_type=jax.ShapeDtypeStruct((N, NH), jnp.float32),
        mesh=mesh,
        compiler_params=pltpu.CompilerParams(needs_layout_passes=False),
        scratch_types=[
            pltpu.VMEM((CHUNK + 16,), jnp.int32),
            pltpu.VMEM((CHUNK + 16,), jnp.int32),
            pltpu.VMEM((CHUNK + 16,), jnp.float32),
            pltpu.VMEM((SUBB, NHP), jnp.int32),
            pltpu.VMEM((SUBB, NHP), jnp.int32),
            pltpu.VMEM((SUBB, NHP), jnp.int32),
            pltpu.VMEM((SUBB, NHP), jnp.int32),
            pltpu.VMEM((SUBB, NHP), jnp.int32),
            pltpu.VMEM((SUBB, NHP), jnp.int32),
            pltpu.VMEM((SUBB, NHP), jnp.int32),
            pltpu.VMEM((SUBB, NHP), jnp.int32),
            pltpu.VMEM((ACC_ROWS, NH), jnp.float32),
            pltpu.SemaphoreType.DMA,
            pltpu.SemaphoreType.DMA,
            pltpu.SemaphoreType.DMA,
            pltpu.SemaphoreType.DMA,
            pltpu.SemaphoreType.DMA,
            pltpu.SemaphoreType.DMA,
            pltpu.SemaphoreType.DMA,
            pltpu.SemaphoreType.DMA,
            pltpu.SemaphoreType.DMA,
        ],
    )
    return f(dst, src, val, table)


def _stage_b_body(r1_ref, r2_ref, w2_ref, x1_ref, x2_ref, h1_ref, h2_ref):
    w2 = w2_ref[...]
    for r_ref, x_ref, h_ref in ((r1_ref, x1_ref, h1_ref),
                                (r2_ref, x2_ref, h2_ref)):
        x = jnp.maximum(r_ref[...], 0.0)
        nrm = jnp.sqrt(jnp.sum(x * x, axis=1, keepdims=True))
        xn = x / jnp.maximum(nrm, 1e-12)
        x_ref[...] = xn
        h = jnp.dot(xn, w2, preferred_element_type=jnp.float32)
        h_ref[...] = h.astype(jnp.bfloat16)


def _stage_b(r1, r2, W2):
    blk = 1000
    grid = (N // blk,)
    row_spec = pl.BlockSpec((blk, NH), lambda i: (i, 0))
    full_spec = pl.BlockSpec((NH, NH), lambda i: (0, 0))
    return pl.pallas_call(
        _stage_b_body,
        grid=grid,
        in_specs=[row_spec, row_spec, full_spec],
        out_specs=[row_spec] * 4,
        out_shape=[jax.ShapeDtypeStruct((N, NH), jnp.float32)] * 2
        + [jax.ShapeDtypeStruct((N, NH), jnp.bfloat16)] * 2,
    )(r1, r2, W2)


def _stage_d_body(r3_ref, r4_ref, x1_ref, x2_ref,
                  w1_ref, b1_ref, w2_ref, b2_ref, w3_ref, b3_ref, out_ref):
    w1, b1 = w1_ref[...], b1_ref[...]
    w2, b2 = w2_ref[...], b2_ref[...]
    w3, b3 = w3_ref[...], b3_ref[...]

    def mlp(t):
        h = jnp.maximum(jnp.dot(t, w1, preferred_element_type=jnp.float32) + b1, 0.0)
        h = jnp.maximum(jnp.dot(h, w2, preferred_element_type=jnp.float32) + b2, 0.0)
        return jnp.dot(h, w3, preferred_element_type=jnp.float32) + b3

    y1 = jnp.maximum(r3_ref[...], 0.0)
    y2 = jnp.maximum(r4_ref[...], 0.0)
    s1 = mlp(x1_ref[...]) + mlp(y1)
    s2 = mlp(x2_ref[...]) + mlp(y2)
    out_ref[...] = s1 * s2


def _stage_d(r3, r4, x1, x2, w1, b1, w2, b2, w3, b3):
    blk = 1000
    grid = (N // blk,)
    row_spec = pl.BlockSpec((blk, NH), lambda i: (i, 0))
    return pl.pallas_call(
        _stage_d_body,
        grid=grid,
        in_specs=[
            row_spec, row_spec, row_spec, row_spec,
            pl.BlockSpec((NH, 2 * NH), lambda i: (0, 0)),
            pl.BlockSpec((1, 2 * NH), lambda i: (0, 0)),
            pl.BlockSpec((2 * NH, 2 * NH), lambda i: (0, 0)),
            pl.BlockSpec((1, 2 * NH), lambda i: (0, 0)),
            pl.BlockSpec((2 * NH, 1), lambda i: (0, 0)),
            pl.BlockSpec((1, 1), lambda i: (0, 0)),
        ],
        out_specs=pl.BlockSpec((blk, 1), lambda i: (i, 0)),
        out_shape=jax.ShapeDtypeStruct((N, 1), jnp.float32),
    )(r3, r4, x1, x2, w1, b1, w2, b2, w3, b3)


def kernel(adj1_indices, adj1_values, adj2_indices, adj2_values,
           W1, W2, mlp_w1, mlp_b1, mlp_w2, mlp_b2, mlp_w3, mlp_b3):
    dst1, src1 = adj1_indices[0], adj1_indices[1]
    dst2, src2 = adj2_indices[0], adj2_indices[1]

    W1p = _pack_table(W1)
    W2p = W2[:, jnp.array(_PERM, dtype=jnp.int32)]
    r1 = _spmm_sc(dst1, src1, adj1_values, W1p)
    r2 = _spmm_sc(dst2, src2, adj2_values, W1p)
    x1, x2, h1b, h2b = _stage_b(r1, r2, W2p)
    h1p = lax.bitcast_convert_type(h1b.reshape(N, NHP, 2), jnp.int32)
    h2p = lax.bitcast_convert_type(h2b.reshape(N, NHP, 2), jnp.int32)
    r3 = _spmm_sc(dst1, src1, adj1_values, h1p)
    r4 = _spmm_sc(dst2, src2, adj2_values, h2p)

    b1 = mlp_b1.reshape(1, -1)
    b2 = mlp_b2.reshape(1, -1)
    b3 = mlp_b3.reshape(1, -1)
    return _stage_d(r3, r4, x1, x2, mlp_w1, b1, mlp_w2, b2, mlp_w3, b3)


# speedup vs baseline: 1.0206x; 1.0206x over previous
import functools

import jax
import jax.numpy as jnp
from jax import lax
from jax.experimental import pallas as pl
from jax.experimental.pallas import tpu as pltpu
from jax.experimental.pallas import tpu_sc as plsc

N = 10000
NH = 256
E = 160000

NC = 2
NS = 16
NW = NC * NS

CHUNK = 3200
SUBB = 16
NRING = 8
ROWS_BASE = 312
BIG_WORKERS = 2
ROWS_BIG = 320
ACC_ROWS = ROWS_BIG

NHP = NH // 2

_PERM = [0] * NH
for _k in range(NH // 32):
    for _i in range(16):
        _PERM[32 * _k + 2 * _i] = 32 * _k + _i
        _PERM[32 * _k + 2 * _i + 1] = 32 * _k + 16 + _i
_PERM = tuple(_PERM)


def _pack_table(t):
    tb = t[:, jnp.array(_PERM, dtype=jnp.int32)].astype(jnp.bfloat16)
    return lax.bitcast_convert_type(tb.reshape(t.shape[0], NHP, 2),
                                    jnp.int32)


def _sc_spmm_body(pk_hbm, val_hbm, table_hbm, out_hbm,
                  pbuf, dbuf, sbuf, vbuf, rows0, rows1, rows2, rows3,
                  rows4, rows5, rows6, rows7, acc,
                  sem, gsem0, gsem1, gsem2, gsem3,
                  gsem4, gsem5, gsem6, gsem7):
    rows_bufs = (rows0, rows1, rows2, rows3, rows4, rows5, rows6, rows7)
    gsems = (gsem0, gsem1, gsem2, gsem3, gsem4, gsem5, gsem6, gsem7)
    wid = lax.axis_index("c") * NS + lax.axis_index("s")
    lo = wid * ROWS_BASE + 8 * jnp.minimum(wid, BIG_WORKERS)
    hi = lo + jnp.where(wid < BIG_WORKERS, ROWS_BIG, ROWS_BASE)

    zf = jnp.zeros((16,), jnp.float32)
    zi = jnp.zeros((16,), jnp.int32)

    def zero_body(r, _):
        for k in range(NH // 16):
            acc[r, pl.ds(k * 16, 16)] = zf
        return 0
    lax.fori_loop(0, ACC_ROWS, zero_body, 0)

    pad_idx = jnp.full((16,), 1, jnp.int32) * lo
    def zero_idx(i, _):
        sbuf[pl.ds(i * 16, 16)] = pad_idx
        return 0
    lax.fori_loop(0, (CHUNK + 16) // 16, zero_idx, 0)

    def chunk_body(ch, _):
        off = ch * CHUNK
        c1 = pltpu.async_copy(pk_hbm.at[pl.ds(off, CHUNK)],
                              pbuf.at[pl.ds(0, CHUNK)], sem)
        c3 = pltpu.async_copy(val_hbm.at[pl.ds(off, CHUNK)],
                              vbuf.at[pl.ds(0, CHUNK)], sem)
        c1.wait()
        c3.wait()

        def comp_body(j, cnt):
            base = j * 32
            p1 = pbuf[pl.ds(base, 16)]
            v1 = vbuf[pl.ds(base, 16)]
            p2 = pbuf[pl.ds(base + 16, 16)]
            v2 = vbuf[pl.ds(base + 16, 16)]
            d1 = lax.shift_right_logical(p1, 16)
            s1 = p1 & 0xFFFF
            d2 = lax.shift_right_logical(p2, 16)
            s2 = p2 & 0xFFFF
            m1 = (d1 >= lo) & (d1 < hi)
            m2 = (d2 >= lo) & (d2 < hi)
            ps1 = plsc.cumsum(m1.astype(jnp.int32))
            ps2 = plsc.cumsum(m2.astype(jnp.int32))
            t1 = ps1[15]
            pos1 = cnt + ps1 - 1
            pos2 = cnt + t1 + ps2 - 1
            plsc.store_scatter(dbuf, [pos1], d1 - lo, mask=m1)
            plsc.store_scatter(sbuf, [pos1], s1, mask=m1)
            plsc.store_scatter(vbuf, [pos1], v1, mask=m1)
            plsc.store_scatter(dbuf, [pos2], d2 - lo, mask=m2)
            plsc.store_scatter(sbuf, [pos2], s2, mask=m2)
            plsc.store_scatter(vbuf, [pos2], v2, mask=m2)
            return cnt + t1 + ps2[15]
        cnt = lax.fori_loop(0, CHUNK // 32, comp_body, 0)

        dbuf[pl.ds(cnt, 16)] = zi
        sbuf[pl.ds(cnt, 16)] = pad_idx
        vbuf[pl.ds(cnt, 16)] = zf

        nbatch = (cnt + SUBB - 1) // SUBB
        nwave = (nbatch + NRING - 1) // NRING

        def wave_body(w, _):
            cps = []
            for r in range(NRING):
                b = w * NRING + r
                cp = pltpu.make_async_copy(
                    table_hbm.at[sbuf.at[pl.ds(b * SUBB, SUBB)]],
                    rows_bufs[r], gsems[r])
                cps.append(cp)

                @pl.when(b < nbatch)
                def _():
                    cp.start()

            for r in range(NRING):
                b = w * NRING + r

                @pl.when(b < nbatch)
                def _():
                    cps[r].wait()
                nedge = jnp.clip(cnt - b * SUBB, 0, SUBB)
                rbuf = rows_bufs[r]

                def e_body(i, _):
                    dle = dbuf[pl.ds(b * SUBB + i, 16)][0]
                    vv = jnp.full((16,), vbuf[pl.ds(b * SUBB + i, 16)][0],
                                  jnp.float32)
                    for k in range(NH // 32):
                        w = rbuf[i, pl.ds(k * 16, 16)]
                        pa, pb = plsc.unpack(
                            plsc.bitcast(w, jnp.bfloat16),
                            format=plsc.PackFormat.INTERLEAVED)
                        plsc.addupdate(acc.at[dle, pl.ds(k * 32, 16)],
                                       pa * vv)
                        plsc.addupdate(acc.at[dle, pl.ds(k * 32 + 16, 16)],
                                       pb * vv)
                    return 0
                lax.fori_loop(0, nedge, e_body, 0)
            return 0
        lax.fori_loop(0, nwave, wave_body, 0)
        return 0

    lax.fori_loop(0, E // CHUNK, chunk_body, 0)

    @pl.when(wid < BIG_WORKERS)
    def _():
        pltpu.sync_copy(acc.at[pl.ds(0, ROWS_BIG), :],
                        out_hbm.at[pl.ds(lo, ROWS_BIG), :])

    @pl.when(wid >= BIG_WORKERS)
    def _():
        pltpu.sync_copy(acc.at[pl.ds(0, ROWS_BASE), :],
                        out_hbm.at[pl.ds(lo, ROWS_BASE), :])


@functools.partial(jax.jit, static_argnums=())
def _spmm_sc(pk, val, table):
    mesh = plsc.VectorSubcoreMesh(core_axis_name="c", subcore_axis_name="s")
    f = pl.kernel(
        _sc_spmm_body,
        out_type=jax.ShapeDtypeStruct((N, NH), jnp.float32),
        mesh=mesh,
        compiler_params=pltpu.CompilerParams(needs_layout_passes=False),
        scratch_types=[
            pltpu.VMEM((CHUNK + 16,), jnp.int32),
            pltpu.VMEM((CHUNK + 16,), jnp.int32),
            pltpu.VMEM((CHUNK + 16,), jnp.int32),
            pltpu.VMEM((CHUNK + 16,), jnp.float32),
            pltpu.VMEM((SUBB, NHP), jnp.int32),
            pltpu.VMEM((SUBB, NHP), jnp.int32),
            pltpu.VMEM((SUBB, NHP), jnp.int32),
            pltpu.VMEM((SUBB, NHP), jnp.int32),
            pltpu.VMEM((SUBB, NHP), jnp.int32),
            pltpu.VMEM((SUBB, NHP), jnp.int32),
            pltpu.VMEM((SUBB, NHP), jnp.int32),
            pltpu.VMEM((SUBB, NHP), jnp.int32),
            pltpu.VMEM((ACC_ROWS, NH), jnp.float32),
            pltpu.SemaphoreType.DMA,
            pltpu.SemaphoreType.DMA,
            pltpu.SemaphoreType.DMA,
            pltpu.SemaphoreType.DMA,
            pltpu.SemaphoreType.DMA,
            pltpu.SemaphoreType.DMA,
            pltpu.SemaphoreType.DMA,
            pltpu.SemaphoreType.DMA,
            pltpu.SemaphoreType.DMA,
        ],
    )
    return f(pk, val, table)


def _pack_edges_body(d_ref, s_ref, o_ref):
    o_ref[...] = d_ref[...] * 65536 + s_ref[...]


def _pack_edges(dst, src):
    d2 = dst.reshape(1250, 128)
    s2 = src.reshape(1250, 128)
    spec = pl.BlockSpec((1250, 128), lambda: (0, 0))
    out = pl.pallas_call(
        _pack_edges_body,
        in_specs=[spec, spec],
        out_specs=spec,
        out_shape=jax.ShapeDtypeStruct((1250, 128), jnp.int32),
    )(d2, s2)
    return out.reshape(E)


def _stage_b_body(r1_ref, r2_ref, w2_ref, x1_ref, x2_ref, h1_ref, h2_ref):
    w2 = w2_ref[...]
    for r_ref, x_ref, h_ref in ((r1_ref, x1_ref, h1_ref),
                                (r2_ref, x2_ref, h2_ref)):
        x = jnp.maximum(r_ref[...], 0.0)
        nrm = jnp.sqrt(jnp.sum(x * x, axis=1, keepdims=True))
        xn = x / jnp.maximum(nrm, 1e-12)
        x_ref[...] = xn
        h = jnp.dot(xn, w2, preferred_element_type=jnp.float32)
        h_ref[...] = h.astype(jnp.bfloat16)


def _stage_b(r1, r2, W2):
    blk = 1000
    grid = (N // blk,)
    row_spec = pl.BlockSpec((blk, NH), lambda i: (i, 0))
    full_spec = pl.BlockSpec((NH, NH), lambda i: (0, 0))
    return pl.pallas_call(
        _stage_b_body,
        grid=grid,
        in_specs=[row_spec, row_spec, full_spec],
        out_specs=[row_spec] * 4,
        out_shape=[jax.ShapeDtypeStruct((N, NH), jnp.float32)] * 2
        + [jax.ShapeDtypeStruct((N, NH), jnp.bfloat16)] * 2,
    )(r1, r2, W2)


def _stage_d_body(r3_ref, r4_ref, x1_ref, x2_ref,
                  w1_ref, b1_ref, w2_ref, b2_ref, w3_ref, b3_ref, out_ref):
    w1, b1 = w1_ref[...], b1_ref[...]
    w2, b2 = w2_ref[...], b2_ref[...]
    w3, b3 = w3_ref[...], b3_ref[...]

    def mlp(t):
        h = jnp.maximum(jnp.dot(t, w1, preferred_element_type=jnp.float32) + b1, 0.0)
        h = jnp.maximum(jnp.dot(h, w2, preferred_element_type=jnp.float32) + b2, 0.0)
        return jnp.dot(h, w3, preferred_element_type=jnp.float32) + b3

    y1 = jnp.maximum(r3_ref[...], 0.0)
    y2 = jnp.maximum(r4_ref[...], 0.0)
    s1 = mlp(x1_ref[...]) + mlp(y1)
    s2 = mlp(x2_ref[...]) + mlp(y2)
    out_ref[...] = s1 * s2


def _stage_d(r3, r4, x1, x2, w1, b1, w2, b2, w3, b3):
    blk = 1000
    grid = (N // blk,)
    row_spec = pl.BlockSpec((blk, NH), lambda i: (i, 0))
    return pl.pallas_call(
        _stage_d_body,
        grid=grid,
        in_specs=[
            row_spec, row_spec, row_spec, row_spec,
            pl.BlockSpec((NH, 2 * NH), lambda i: (0, 0)),
            pl.BlockSpec((1, 2 * NH), lambda i: (0, 0)),
            pl.BlockSpec((2 * NH, 2 * NH), lambda i: (0, 0)),
            pl.BlockSpec((1, 2 * NH), lambda i: (0, 0)),
            pl.BlockSpec((2 * NH, 1), lambda i: (0, 0)),
            pl.BlockSpec((1, 1), lambda i: (0, 0)),
        ],
        out_specs=pl.BlockSpec((blk, 1), lambda i: (i, 0)),
        out_shape=jax.ShapeDtypeStruct((N, 1), jnp.float32),
    )(r3, r4, x1, x2, w1, b1, w2, b2, w3, b3)


def kernel(adj1_indices, adj1_values, adj2_indices, adj2_values,
           W1, W2, mlp_w1, mlp_b1, mlp_w2, mlp_b2, mlp_w3, mlp_b3):
    dst1, src1 = adj1_indices[0], adj1_indices[1]
    dst2, src2 = adj2_indices[0], adj2_indices[1]

    W1p = _pack_table(W1)
    W2p = W2[:, jnp.array(_PERM, dtype=jnp.int32)]
    pk1 = _pack_edges(dst1, src1)
    pk2 = _pack_edges(dst2, src2)
    r1 = _spmm_sc(pk1, adj1_values, W1p)
    r2 = _spmm_sc(pk2, adj2_values, W1p)
    x1, x2, h1b, h2b = _stage_b(r1, r2, W2p)
    h1p = lax.bitcast_convert_type(h1b.reshape(N, NHP, 2), jnp.int32)
    h2p = lax.bitcast_convert_type(h2b.reshape(N, NHP, 2), jnp.int32)
    r3 = _spmm_sc(pk1, adj1_values, h1p)
    r4 = _spmm_sc(pk2, adj2_values, h2p)

    b1 = mlp_b1.reshape(1, -1)
    b2 = mlp_b2.reshape(1, -1)
    b3 = mlp_b3.reshape(1, -1)
    return _stage_d(r3, r4, x1, x2, mlp_w1, b1, mlp_w2, b2, mlp_w3, b3)


# speedup vs baseline: 1.0475x; 1.0264x over previous
import functools

import jax
import jax.numpy as jnp
from jax import lax
from jax.experimental import pallas as pl
from jax.experimental.pallas import tpu as pltpu
from jax.experimental.pallas import tpu_sc as plsc

N = 10000
NH = 256
E = 160000

NC = 2
NS = 16
NW = NC * NS

CHUNK = 6400
SUBB = 16
NRING = 8
ROWS_BASE = 312
BIG_WORKERS = 2
ROWS_BIG = 320
ACC_ROWS = ROWS_BIG

NHP = NH // 2

_PERM = [0] * NH
for _k in range(NH // 32):
    for _i in range(16):
        _PERM[32 * _k + 2 * _i] = 32 * _k + _i
        _PERM[32 * _k + 2 * _i + 1] = 32 * _k + 16 + _i
_PERM = tuple(_PERM)


def _pack_table(t):
    tb = t[:, jnp.array(_PERM, dtype=jnp.int32)].astype(jnp.bfloat16)
    return lax.bitcast_convert_type(tb.reshape(t.shape[0], NHP, 2),
                                    jnp.int32)


def _sc_spmm_body(pk_hbm, val_hbm, table_hbm, out_hbm,
                  pbuf, dbuf, sbuf, vbuf, rows0, rows1, rows2, rows3,
                  rows4, rows5, rows6, rows7, acc,
                  sem, gsem0, gsem1, gsem2, gsem3,
                  gsem4, gsem5, gsem6, gsem7):
    rows_bufs = (rows0, rows1, rows2, rows3, rows4, rows5, rows6, rows7)
    gsems = (gsem0, gsem1, gsem2, gsem3, gsem4, gsem5, gsem6, gsem7)
    wid = lax.axis_index("c") * NS + lax.axis_index("s")
    lo = wid * ROWS_BASE + 8 * jnp.minimum(wid, BIG_WORKERS)
    hi = lo + jnp.where(wid < BIG_WORKERS, ROWS_BIG, ROWS_BASE)

    zf = jnp.zeros((16,), jnp.float32)
    zi = jnp.zeros((16,), jnp.int32)

    def zero_body(r, _):
        for k in range(NH // 16):
            acc[r, pl.ds(k * 16, 16)] = zf
        return 0
    lax.fori_loop(0, ACC_ROWS, zero_body, 0)

    pad_idx = jnp.full((16,), 1, jnp.int32) * lo
    def zero_idx(i, _):
        sbuf[pl.ds(i * 16, 16)] = pad_idx
        return 0
    lax.fori_loop(0, (CHUNK + 16) // 16, zero_idx, 0)

    def chunk_body(ch, _):
        off = ch * CHUNK
        c1 = pltpu.async_copy(pk_hbm.at[pl.ds(off, CHUNK)],
                              pbuf.at[pl.ds(0, CHUNK)], sem)
        c3 = pltpu.async_copy(val_hbm.at[pl.ds(off, CHUNK)],
                              vbuf.at[pl.ds(0, CHUNK)], sem)
        c1.wait()
        c3.wait()

        def comp_body(j, cnt):
            base = j * 32
            p1 = pbuf[pl.ds(base, 16)]
            v1 = vbuf[pl.ds(base, 16)]
            p2 = pbuf[pl.ds(base + 16, 16)]
            v2 = vbuf[pl.ds(base + 16, 16)]
            d1 = lax.shift_right_logical(p1, 16)
            s1 = p1 & 0xFFFF
            d2 = lax.shift_right_logical(p2, 16)
            s2 = p2 & 0xFFFF
            m1 = (d1 >= lo) & (d1 < hi)
            m2 = (d2 >= lo) & (d2 < hi)
            ps1 = plsc.cumsum(m1.astype(jnp.int32))
            ps2 = plsc.cumsum(m2.astype(jnp.int32))
            t1 = ps1[15]
            pos1 = cnt + ps1 - 1
            pos2 = cnt + t1 + ps2 - 1
            plsc.store_scatter(dbuf, [pos1], d1 - lo, mask=m1)
            plsc.store_scatter(sbuf, [pos1], s1, mask=m1)
            plsc.store_scatter(vbuf, [pos1], v1, mask=m1)
            plsc.store_scatter(dbuf, [pos2], d2 - lo, mask=m2)
            plsc.store_scatter(sbuf, [pos2], s2, mask=m2)
            plsc.store_scatter(vbuf, [pos2], v2, mask=m2)
            return cnt + t1 + ps2[15]
        cnt = lax.fori_loop(0, CHUNK // 32, comp_body, 0)

        dbuf[pl.ds(cnt, 16)] = zi
        sbuf[pl.ds(cnt, 16)] = pad_idx
        vbuf[pl.ds(cnt, 16)] = zf

        nbatch = (cnt + SUBB - 1) // SUBB
        nwave = (nbatch + NRING - 1) // NRING

        def wave_body(w, _):
            cps = []
            for r in range(NRING):
                b = w * NRING + r
                cp = pltpu.make_async_copy(
                    table_hbm.at[sbuf.at[pl.ds(b * SUBB, SUBB)]],
                    rows_bufs[r], gsems[r])
                cps.append(cp)

                @pl.when(b < nbatch)
                def _():
                    cp.start()

            for r in range(NRING):
                b = w * NRING + r

                @pl.when(b < nbatch)
                def _():
                    cps[r].wait()
                nedge = jnp.clip(cnt - b * SUBB, 0, SUBB)
                rbuf = rows_bufs[r]

                def e_body(i, _):
                    dle = dbuf[pl.ds(b * SUBB + i, 16)][0]
                    vv = jnp.full((16,), vbuf[pl.ds(b * SUBB + i, 16)][0],
                                  jnp.float32)
                    for k in range(NH // 32):
                        w = rbuf[i, pl.ds(k * 16, 16)]
                        pa, pb = plsc.unpack(
                            plsc.bitcast(w, jnp.bfloat16),
                            format=plsc.PackFormat.INTERLEAVED)
                        plsc.addupdate(acc.at[dle, pl.ds(k * 32, 16)],
                                       pa * vv)
                        plsc.addupdate(acc.at[dle, pl.ds(k * 32 + 16, 16)],
                                       pb * vv)
                    return 0
                lax.fori_loop(0, nedge, e_body, 0)
            return 0
        lax.fori_loop(0, nwave, wave_body, 0)
        return 0

    lax.fori_loop(0, E // CHUNK, chunk_body, 0)

    @pl.when(wid < BIG_WORKERS)
    def _():
        pltpu.sync_copy(acc.at[pl.ds(0, ROWS_BIG), :],
                        out_hbm.at[pl.ds(lo, ROWS_BIG), :])

    @pl.when(wid >= BIG_WORKERS)
    def _():
        pltpu.sync_copy(acc.at[pl.ds(0, ROWS_BASE), :],
                        out_hbm.at[pl.ds(lo, ROWS_BASE), :])


@functools.partial(jax.jit, static_argnums=())
def _spmm_sc(pk, val, table):
    mesh = plsc.VectorSubcoreMesh(core_axis_name="c", subcore_axis_name="s")
    f = pl.kernel(
        _sc_spmm_body,
        out_type=jax.ShapeDtypeStruct((N, NH), jnp.float32),
        mesh=mesh,
        compiler_params=pltpu.CompilerParams(needs_layout_passes=False),
        scratch_types=[
            pltpu.VMEM((CHUNK + 16,), jnp.int32),
            pltpu.VMEM((CHUNK + 16,), jnp.int32),
            pltpu.VMEM((CHUNK + 16,), jnp.int32),
            pltpu.VMEM((CHUNK + 16,), jnp.float32),
            pltpu.VMEM((SUBB, NHP), jnp.int32),
            pltpu.VMEM((SUBB, NHP), jnp.int32),
            pltpu.VMEM((SUBB, NHP), jnp.int32),
            pltpu.VMEM((SUBB, NHP), jnp.int32),
            pltpu.VMEM((SUBB, NHP), jnp.int32),
            pltpu.VMEM((SUBB, NHP), jnp.int32),
            pltpu.VMEM((SUBB, NHP), jnp.int32),
            pltpu.VMEM((SUBB, NHP), jnp.int32),
            pltpu.VMEM((ACC_ROWS, NH), jnp.float32),
            pltpu.SemaphoreType.DMA,
            pltpu.SemaphoreType.DMA,
            pltpu.SemaphoreType.DMA,
            pltpu.SemaphoreType.DMA,
            pltpu.SemaphoreType.DMA,
            pltpu.SemaphoreType.DMA,
            pltpu.SemaphoreType.DMA,
            pltpu.SemaphoreType.DMA,
            pltpu.SemaphoreType.DMA,
        ],
    )
    return f(pk, val, table)


def _pack_edges_body(d_ref, s_ref, o_ref):
    o_ref[...] = d_ref[...] * 65536 + s_ref[...]


def _pack_edges(dst, src):
    d2 = dst.reshape(1250, 128)
    s2 = src.reshape(1250, 128)
    spec = pl.BlockSpec((1250, 128), lambda: (0, 0))
    out = pl.pallas_call(
        _pack_edges_body,
        in_specs=[spec, spec],
        out_specs=spec,
        out_shape=jax.ShapeDtypeStruct((1250, 128), jnp.int32),
    )(d2, s2)
    return out.reshape(E)


def _stage_b_body(r1_ref, r2_ref, w2_ref, x1_ref, x2_ref, h1_ref, h2_ref):
    w2 = w2_ref[...]
    for r_ref, x_ref, h_ref in ((r1_ref, x1_ref, h1_ref),
                                (r2_ref, x2_ref, h2_ref)):
        x = jnp.maximum(r_ref[...], 0.0)
        nrm = jnp.sqrt(jnp.sum(x * x, axis=1, keepdims=True))
        xn = x / jnp.maximum(nrm, 1e-12)
        x_ref[...] = xn
        h = jnp.dot(xn, w2, preferred_element_type=jnp.float32)
        h_ref[...] = h.astype(jnp.bfloat16)


def _stage_b(r1, r2, W2):
    blk = 1000
    grid = (N // blk,)
    row_spec = pl.BlockSpec((blk, NH), lambda i: (i, 0))
    full_spec = pl.BlockSpec((NH, NH), lambda i: (0, 0))
    return pl.pallas_call(
        _stage_b_body,
        grid=grid,
        in_specs=[row_spec, row_spec, full_spec],
        out_specs=[row_spec] * 4,
        out_shape=[jax.ShapeDtypeStruct((N, NH), jnp.float32)] * 2
        + [jax.ShapeDtypeStruct((N, NH), jnp.bfloat16)] * 2,
    )(r1, r2, W2)


def _stage_d_body(r3_ref, r4_ref, x1_ref, x2_ref,
                  w1_ref, b1_ref, w2_ref, b2_ref, w3_ref, b3_ref, out_ref):
    w1, b1 = w1_ref[...], b1_ref[...]
    w2, b2 = w2_ref[...], b2_ref[...]
    w3, b3 = w3_ref[...], b3_ref[...]

    def mlp(t):
        h = jnp.maximum(jnp.dot(t, w1, preferred_element_type=jnp.float32) + b1, 0.0)
        h = jnp.maximum(jnp.dot(h, w2, preferred_element_type=jnp.float32) + b2, 0.0)
        return jnp.dot(h, w3, preferred_element_type=jnp.float32) + b3

    y1 = jnp.maximum(r3_ref[...], 0.0)
    y2 = jnp.maximum(r4_ref[...], 0.0)
    s1 = mlp(x1_ref[...]) + mlp(y1)
    s2 = mlp(x2_ref[...]) + mlp(y2)
    out_ref[...] = s1 * s2


def _stage_d(r3, r4, x1, x2, w1, b1, w2, b2, w3, b3):
    blk = 1000
    grid = (N // blk,)
    row_spec = pl.BlockSpec((blk, NH), lambda i: (i, 0))
    return pl.pallas_call(
        _stage_d_body,
        grid=grid,
        in_specs=[
            row_spec, row_spec, row_spec, row_spec,
            pl.BlockSpec((NH, 2 * NH), lambda i: (0, 0)),
            pl.BlockSpec((1, 2 * NH), lambda i: (0, 0)),
            pl.BlockSpec((2 * NH, 2 * NH), lambda i: (0, 0)),
            pl.BlockSpec((1, 2 * NH), lambda i: (0, 0)),
            pl.BlockSpec((2 * NH, 1), lambda i: (0, 0)),
            pl.BlockSpec((1, 1), lambda i: (0, 0)),
        ],
        out_specs=pl.BlockSpec((blk, 1), lambda i: (i, 0)),
        out_shape=jax.ShapeDtypeStruct((N, 1), jnp.float32),
    )(r3, r4, x1, x2, w1, b1, w2, b2, w3, b3)


def kernel(adj1_indices, adj1_values, adj2_indices, adj2_values,
           W1, W2, mlp_w1, mlp_b1, mlp_w2, mlp_b2, mlp_w3, mlp_b3):
    dst1, src1 = adj1_indices[0], adj1_indices[1]
    dst2, src2 = adj2_indices[0], adj2_indices[1]

    W1p = _pack_table(W1)
    W2p = W2[:, jnp.array(_PERM, dtype=jnp.int32)]
    pk1 = _pack_edges(dst1, src1)
    pk2 = _pack_edges(dst2, src2)
    r1 = _spmm_sc(pk1, adj1_values, W1p)
    r2 = _spmm_sc(pk2, adj2_values, W1p)
    x1, x2, h1b, h2b = _stage_b(r1, r2, W2p)
    h1p = lax.bitcast_convert_type(h1b.reshape(N, NHP, 2), jnp.int32)
    h2p = lax.bitcast_convert_type(h2b.reshape(N, NHP, 2), jnp.int32)
    r3 = _spmm_sc(pk1, adj1_values, h1p)
    r4 = _spmm_sc(pk2, adj2_values, h2p)

    b1 = mlp_b1.reshape(1, -1)
    b2 = mlp_b2.reshape(1, -1)
    b3 = mlp_b3.reshape(1, -1)
    return _stage_d(r3, r4, x1, x2, mlp_w1, b1, mlp_w2, b2, mlp_w3, b3)
